# Initial kernel scaffold; baseline (speedup 1.0000x reference)
import jax
import jax.numpy as jnp
from jax.experimental import pallas as pl


def kernel(edge_vec, edge_idx, pred_energy, atomic_stress_iso, atomic_stress_aniso, cell_volume, num_atoms, batch):
    raise NotImplementedError("write your pallas kernel here")


# speedup vs baseline: 1.7829x; 1.7829x over previous
import math

import jax
import jax.numpy as jnp
import numpy as np
from jax import lax
from jax.experimental import pallas as pl
from jax.experimental.pallas import tpu as pltpu
from jax.experimental.pallas import tpu_sc as plsc

N_ATOMS = 100000
N_EDGES = 3200000
N_BATCH = 16

NC = 2
NS = 16
NW = NC * NS

G128 = 128
N_GROUPS = N_EDGES // G128
CHUNK_G = 8
CHUNK = CHUNK_G * G128

MAIN_CHUNKS = 97
MAIN_G = MAIN_CHUNKS * CHUNK_G
EXTRA_BASE_G = NW * MAIN_G
N_EXTRA = 21

ROWS_A = 6248
ROWS_LAST = N_ATOMS - 15 * ROWS_A


def _sc_edge_body(ev, src2, dst2, z8, f_out, v_out,
                  f_acc, v_acc, vec8b, nvec8b, vir8b, srcb, dstb):
    c = lax.axis_index("c")
    s = lax.axis_index("s")
    w = s * NC + c

    r0 = s * ROWS_A
    pltpu.sync_copy(z8, f_acc.at[pl.ds(r0, ROWS_LAST), :])
    pltpu.sync_copy(z8, v_acc.at[pl.ds(r0, ROWS_LAST), :])
    pltpu.sync_copy(z8.at[pl.ds(0, CHUNK), :], vec8b)
    pltpu.sync_copy(z8.at[pl.ds(0, CHUNK), :], nvec8b)
    pltpu.sync_copy(z8.at[pl.ds(0, CHUNK), :], vir8b)
    plsc.subcore_barrier()

    iota = lax.iota(jnp.int32, 16)
    cols = [jnp.full((16,), j, jnp.int32) for j in range(6)]
    neg1 = jnp.full((16,), -1.0, jnp.float32)

    def do_chunk(gbase, ngroups):
        n = ngroups * G128
        e0 = gbase * G128
        pltpu.sync_copy(ev.at[pl.ds(e0, n), :],
                        vec8b.at[pl.ds(0, n), pl.ds(0, 3)])
        pltpu.sync_copy(src2.at[pl.ds(gbase, ngroups), :],
                        srcb.at[pl.ds(0, ngroups), :])
        pltpu.sync_copy(dst2.at[pl.ds(gbase, ngroups), :],
                        dstb.at[pl.ds(0, ngroups), :])

        def cbody(i, _):
            rows = i * 16 + iota
            x = plsc.load_gather(vec8b, [rows, cols[0]])
            y = plsc.load_gather(vec8b, [rows, cols[1]])
            z = plsc.load_gather(vec8b, [rows, cols[2]])
            plsc.store_scatter(nvec8b, [rows, cols[0]], x * neg1)
            plsc.store_scatter(nvec8b, [rows, cols[1]], y * neg1)
            plsc.store_scatter(nvec8b, [rows, cols[2]], z * neg1)
            plsc.store_scatter(vir8b, [rows, cols[0]], x * x)
            plsc.store_scatter(vir8b, [rows, cols[1]], y * y)
            plsc.store_scatter(vir8b, [rows, cols[2]], z * z)
            plsc.store_scatter(vir8b, [rows, cols[3]], x * y)
            plsc.store_scatter(vir8b, [rows, cols[4]], y * z)
            plsc.store_scatter(vir8b, [rows, cols[5]], z * x)
            return _

        lax.fori_loop(0, n // 16, cbody, 0, unroll=4)

        def sbody(g, _):
            rows = pl.ds(g * G128, G128)
            pltpu.sync_copy(vec8b.at[rows, :], f_acc.at[srcb.at[g]], add=True)
            pltpu.sync_copy(nvec8b.at[rows, :], f_acc.at[dstb.at[g]], add=True)
            pltpu.sync_copy(vir8b.at[rows, :], v_acc.at[dstb.at[g]], add=True)
            return _

        lax.fori_loop(0, ngroups, sbody, 0)

    def mbody(k, _):
        do_chunk(w * MAIN_G + k * CHUNK_G, CHUNK_G)
        return _

    lax.fori_loop(0, MAIN_CHUNKS, mbody, 0)

    @pl.when(w < N_EXTRA)
    def _():
        do_chunk(EXTRA_BASE_G + w * CHUNK_G, CHUNK_G)

    plsc.subcore_barrier()

    @pl.when(s < 15)
    def _():
        rows = pl.ds(s * ROWS_A, ROWS_A)
        pltpu.sync_copy(f_acc.at[rows, :], f_out.at[c, rows, :])
        pltpu.sync_copy(v_acc.at[rows, :], v_out.at[c, rows, :])

    @pl.when(s == 15)
    def _():
        rows = pl.ds(15 * ROWS_A, ROWS_LAST)
        pltpu.sync_copy(f_acc.at[rows, :], f_out.at[c, rows, :])
        pltpu.sync_copy(v_acc.at[rows, :], v_out.at[c, rows, :])


@jax.jit
def _sc_edge_scatter(edge_vec, src2, dst2):
    z8 = jnp.zeros((ROWS_LAST, 8), jnp.float32)
    f = pl.kernel(
        _sc_edge_body,
        out_type=(
            jax.ShapeDtypeStruct((NC, N_ATOMS, 8), jnp.float32),
            jax.ShapeDtypeStruct((NC, N_ATOMS, 8), jnp.float32),
        ),
        mesh=plsc.VectorSubcoreMesh(
            core_axis_name="c", subcore_axis_name="s",
            num_cores=NC, num_subcores=NS),
        compiler_params=pltpu.CompilerParams(
            needs_layout_passes=False, use_tc_tiling_on_sc=False),
        scratch_types=[
            pltpu.VMEM_SHARED((N_ATOMS, 8), jnp.float32),
            pltpu.VMEM_SHARED((N_ATOMS, 8), jnp.float32),
            pltpu.VMEM((CHUNK, 8), jnp.float32),
            pltpu.VMEM((CHUNK, 8), jnp.float32),
            pltpu.VMEM((CHUNK, 8), jnp.float32),
            pltpu.VMEM((CHUNK_G, G128), jnp.int32),
            pltpu.VMEM((CHUNK_G, G128), jnp.int32),
        ],
    )
    return f(edge_vec, src2, dst2, z8)


_A_T = np.array([
    [0.0, 0.0, 0.0, 0.0, 0.0, 1 / math.sqrt(2)],
    [0.0, 0.0, 0.0, 1 / math.sqrt(2), 0.0, 0.0],
    [-1 / math.sqrt(6), math.sqrt(2) / math.sqrt(3), -1 / math.sqrt(6), 0.0, 0.0, 0.0],
    [0.0, 0.0, 0.0, 0.0, 1 / math.sqrt(2), 0.0],
    [-1 / math.sqrt(2), 0.0, 1 / math.sqrt(2), 0.0, 0.0, 0.0],
], dtype=np.float32)

_BLK = 1000
_NBLK = N_ATOMS // _BLK


def _tc_combine_body(fv, vv, batch3, iso, aniso, vol, a_t,
                     force_o, stress_o, andev_o, dev_o):
    i = pl.program_id(0)
    fvb = fv[...]
    vvb = vv[...]
    force_o[...] = fvb[0, :, 0:3] + fvb[1, :, 0:3]

    b = batch3[0]
    oh = (lax.broadcasted_iota(jnp.int32, (N_BATCH, _BLK), 0) == b
          ).astype(jnp.float32)
    sp = jnp.dot(oh, vvb[0] + vvb[1], preferred_element_type=jnp.float32)
    ap = jnp.dot(jnp.dot(oh, aniso[...], preferred_element_type=jnp.float32),
                 a_t[...], preferred_element_type=jnp.float32)
    ip = jnp.dot(oh, iso[...], preferred_element_type=jnp.float32)
    colmask = (lax.broadcasted_iota(jnp.int32, (N_BATCH, 6), 1) < 3
               ).astype(jnp.float32)

    @pl.when(i == 0)
    def _():
        stress_o[...] = jnp.zeros_like(stress_o)
        andev_o[...] = jnp.zeros_like(andev_o)
        dev_o[...] = jnp.zeros_like(dev_o)

    stress_o[...] += sp[:, 0:6]
    andev_o[...] += ap
    dev_o[...] += ip * colmask

    @pl.when(i == _NBLK - 1)
    def _():
        stress_o[...] = stress_o[...] * (-1.0 / vol[...])


@jax.jit
def _tc_combine(fv, vv, batch3, iso, aniso, vol2):
    return pl.pallas_call(
        _tc_combine_body,
        grid=(_NBLK,),
        in_specs=[
            pl.BlockSpec((NC, _BLK, 8), lambda i: (0, i, 0)),
            pl.BlockSpec((NC, _BLK, 8), lambda i: (0, i, 0)),
            pl.BlockSpec((1, 1, _BLK), lambda i: (i, 0, 0)),
            pl.BlockSpec((_BLK, 1), lambda i: (i, 0)),
            pl.BlockSpec((_BLK, 5), lambda i: (i, 0)),
            pl.BlockSpec((N_BATCH, 1), lambda i: (0, 0)),
            pl.BlockSpec((5, 6), lambda i: (0, 0)),
        ],
        out_specs=[
            pl.BlockSpec((_BLK, 3), lambda i: (i, 0)),
            pl.BlockSpec((N_BATCH, 6), lambda i: (0, 0)),
            pl.BlockSpec((N_BATCH, 6), lambda i: (0, 0)),
            pl.BlockSpec((N_BATCH, 6), lambda i: (0, 0)),
        ],
        out_shape=[
            jax.ShapeDtypeStruct((N_ATOMS, 3), jnp.float32),
            jax.ShapeDtypeStruct((N_BATCH, 6), jnp.float32),
            jax.ShapeDtypeStruct((N_BATCH, 6), jnp.float32),
            jax.ShapeDtypeStruct((N_BATCH, 6), jnp.float32),
        ],
    )(fv, vv, batch3, iso, aniso, vol2, _A_T)


def kernel(edge_vec, edge_idx, pred_energy, atomic_stress_iso,
           atomic_stress_aniso, cell_volume, num_atoms, batch):
    src2 = edge_idx[0].reshape(N_GROUPS, G128)
    dst2 = edge_idx[1].reshape(N_GROUPS, G128)
    fv, vv = _sc_edge_scatter(edge_vec, src2, dst2)

    batch3 = batch.reshape(_NBLK, 1, _BLK)
    vol2 = cell_volume.reshape(N_BATCH, 1)
    force_drv, stress_drv, output_andev, output_dev = _tc_combine(
        fv, vv, batch3, atomic_stress_iso, atomic_stress_aniso, vol2)

    energy_out = jnp.squeeze(pred_energy, -1)
    return (energy_out, force_drv, stress_drv, output_andev, output_dev)


# speedup vs baseline: 17.5129x; 9.8229x over previous
import math

import jax
import jax.numpy as jnp
import numpy as np
from jax import lax
from jax.experimental import pallas as pl
from jax.experimental.pallas import tpu as pltpu
from jax.experimental.pallas import tpu_sc as plsc

N_ATOMS = 100000
N_EDGES = 3200000
N_BATCH = 16

NC = 2
NS = 16
NW = NC * NS

G128 = 128
N_GROUPS = N_EDGES // G128
CHUNK_G = 4
CHUNK = CHUNK_G * G128

MAIN_CHUNKS = 195
MAIN_G = MAIN_CHUNKS * CHUNK_G
EXTRA_BASE_G = NW * MAIN_G
N_EXTRA = 10

ROWS_A = 6248
ROWS_LAST = N_ATOMS - 15 * ROWS_A


def _sc_edge_body(evx, evy, evz, src2, dst2, z8, f_out, v_out,
                  f_acc, v_acc, xb, yb, zb, vec8b, nvec8b, vir8b, srcb, dstb):
    c = lax.axis_index("c")
    s = lax.axis_index("s")
    w = s * NC + c

    r0 = s * ROWS_A
    pltpu.sync_copy(z8, f_acc.at[pl.ds(r0, ROWS_LAST), :])
    pltpu.sync_copy(z8, v_acc.at[pl.ds(r0, ROWS_LAST), :])
    pltpu.sync_copy(z8.at[pl.ds(0, CHUNK), :], vec8b)
    pltpu.sync_copy(z8.at[pl.ds(0, CHUNK), :], nvec8b)
    pltpu.sync_copy(z8.at[pl.ds(0, CHUNK), :], vir8b)
    plsc.subcore_barrier()

    iota = lax.iota(jnp.int32, 16)
    cols = [jnp.full((16,), j, jnp.int32) for j in range(6)]
    neg1 = jnp.full((16,), -1.0, jnp.float32)

    def do_chunk(gbase, ngroups):
        n = ngroups * G128
        e0 = gbase * G128
        pltpu.sync_copy(evx.at[pl.ds(e0, n)], xb.at[pl.ds(0, n)])
        pltpu.sync_copy(evy.at[pl.ds(e0, n)], yb.at[pl.ds(0, n)])
        pltpu.sync_copy(evz.at[pl.ds(e0, n)], zb.at[pl.ds(0, n)])
        pltpu.sync_copy(src2.at[pl.ds(gbase, ngroups), :],
                        srcb.at[pl.ds(0, ngroups), :])
        pltpu.sync_copy(dst2.at[pl.ds(gbase, ngroups), :],
                        dstb.at[pl.ds(0, ngroups), :])

        def cbody(i, _):
            rows = i * 16 + iota
            x = plsc.load_gather(xb, [rows])
            y = plsc.load_gather(yb, [rows])
            z = plsc.load_gather(zb, [rows])
            plsc.store_scatter(vec8b, [rows, cols[0]], x)
            plsc.store_scatter(vec8b, [rows, cols[1]], y)
            plsc.store_scatter(vec8b, [rows, cols[2]], z)
            plsc.store_scatter(nvec8b, [rows, cols[0]], x * neg1)
            plsc.store_scatter(nvec8b, [rows, cols[1]], y * neg1)
            plsc.store_scatter(nvec8b, [rows, cols[2]], z * neg1)
            plsc.store_scatter(vir8b, [rows, cols[0]], x * x)
            plsc.store_scatter(vir8b, [rows, cols[1]], y * y)
            plsc.store_scatter(vir8b, [rows, cols[2]], z * z)
            plsc.store_scatter(vir8b, [rows, cols[3]], x * y)
            plsc.store_scatter(vir8b, [rows, cols[4]], y * z)
            plsc.store_scatter(vir8b, [rows, cols[5]], z * x)
            return _

        lax.fori_loop(0, n // 16, cbody, 0, unroll=4)

        def sbody(g, _):
            rows = pl.ds(g * G128, G128)
            pltpu.sync_copy(vec8b.at[rows, :], f_acc.at[srcb.at[g]], add=True)
            pltpu.sync_copy(nvec8b.at[rows, :], f_acc.at[dstb.at[g]], add=True)
            pltpu.sync_copy(vir8b.at[rows, :], v_acc.at[dstb.at[g]], add=True)
            return _

        lax.fori_loop(0, ngroups, sbody, 0)

    def mbody(k, _):
        do_chunk(w * MAIN_G + k * CHUNK_G, CHUNK_G)
        return _

    lax.fori_loop(0, MAIN_CHUNKS, mbody, 0)

    @pl.when(w < N_EXTRA)
    def _():
        do_chunk(EXTRA_BASE_G + w * CHUNK_G, CHUNK_G)

    plsc.subcore_barrier()

    @pl.when(s < 15)
    def _():
        rows = pl.ds(s * ROWS_A, ROWS_A)
        pltpu.sync_copy(f_acc.at[rows, :], f_out.at[c, rows, :])
        pltpu.sync_copy(v_acc.at[rows, :], v_out.at[c, rows, :])

    @pl.when(s == 15)
    def _():
        rows = pl.ds(15 * ROWS_A, ROWS_LAST)
        pltpu.sync_copy(f_acc.at[rows, :], f_out.at[c, rows, :])
        pltpu.sync_copy(v_acc.at[rows, :], v_out.at[c, rows, :])


@jax.jit
def _sc_edge_scatter(evx, evy, evz, src2, dst2):
    z8 = jnp.zeros((ROWS_LAST, 8), jnp.float32)
    f = pl.kernel(
        _sc_edge_body,
        out_type=(
            jax.ShapeDtypeStruct((NC, N_ATOMS, 8), jnp.float32),
            jax.ShapeDtypeStruct((NC, N_ATOMS, 8), jnp.float32),
        ),
        mesh=plsc.VectorSubcoreMesh(
            core_axis_name="c", subcore_axis_name="s",
            num_cores=NC, num_subcores=NS),
        compiler_params=pltpu.CompilerParams(
            needs_layout_passes=False, use_tc_tiling_on_sc=False),
        scratch_types=[
            pltpu.VMEM_SHARED((N_ATOMS, 8), jnp.float32),
            pltpu.VMEM_SHARED((N_ATOMS, 8), jnp.float32),
            pltpu.VMEM((CHUNK,), jnp.float32),
            pltpu.VMEM((CHUNK,), jnp.float32),
            pltpu.VMEM((CHUNK,), jnp.float32),
            pltpu.VMEM((CHUNK, 8), jnp.float32),
            pltpu.VMEM((CHUNK, 8), jnp.float32),
            pltpu.VMEM((CHUNK, 8), jnp.float32),
            pltpu.VMEM((CHUNK_G, G128), jnp.int32),
            pltpu.VMEM((CHUNK_G, G128), jnp.int32),
        ],
    )
    return f(evx, evy, evz, src2, dst2, z8)


_A_T = np.array([
    [0.0, 0.0, 0.0, 0.0, 0.0, 1 / math.sqrt(2)],
    [0.0, 0.0, 0.0, 1 / math.sqrt(2), 0.0, 0.0],
    [-1 / math.sqrt(6), math.sqrt(2) / math.sqrt(3), -1 / math.sqrt(6), 0.0, 0.0, 0.0],
    [0.0, 0.0, 0.0, 0.0, 1 / math.sqrt(2), 0.0],
    [-1 / math.sqrt(2), 0.0, 1 / math.sqrt(2), 0.0, 0.0, 0.0],
], dtype=np.float32)

_BLK = 1000
_NBLK = N_ATOMS // _BLK


def _tc_combine_body(fv, vv, batch3, iso, aniso, vol, a_t,
                     force_o, stress_o, andev_o, dev_o):
    i = pl.program_id(0)
    fvb = fv[...]
    vvb = vv[...]
    force_o[...] = fvb[0, :, 0:3] + fvb[1, :, 0:3]

    b = batch3[0]
    oh = (lax.broadcasted_iota(jnp.int32, (N_BATCH, _BLK), 0) == b
          ).astype(jnp.float32)
    sp = jnp.dot(oh, vvb[0] + vvb[1], preferred_element_type=jnp.float32)
    ap = jnp.dot(jnp.dot(oh, aniso[...], preferred_element_type=jnp.float32),
                 a_t[...], preferred_element_type=jnp.float32)
    ip = jnp.dot(oh, iso[...], preferred_element_type=jnp.float32)
    colmask = (lax.broadcasted_iota(jnp.int32, (N_BATCH, 6), 1) < 3
               ).astype(jnp.float32)

    @pl.when(i == 0)
    def _():
        stress_o[...] = jnp.zeros_like(stress_o)
        andev_o[...] = jnp.zeros_like(andev_o)
        dev_o[...] = jnp.zeros_like(dev_o)

    stress_o[...] += sp[:, 0:6]
    andev_o[...] += ap
    dev_o[...] += ip * colmask

    @pl.when(i == _NBLK - 1)
    def _():
        stress_o[...] = stress_o[...] * (-1.0 / vol[...])


@jax.jit
def _tc_combine(fv, vv, batch3, iso, aniso, vol2):
    return pl.pallas_call(
        _tc_combine_body,
        grid=(_NBLK,),
        in_specs=[
            pl.BlockSpec((NC, _BLK, 8), lambda i: (0, i, 0)),
            pl.BlockSpec((NC, _BLK, 8), lambda i: (0, i, 0)),
            pl.BlockSpec((1, 1, _BLK), lambda i: (i, 0, 0)),
            pl.BlockSpec((_BLK, 1), lambda i: (i, 0)),
            pl.BlockSpec((_BLK, 5), lambda i: (i, 0)),
            pl.BlockSpec((N_BATCH, 1), lambda i: (0, 0)),
            pl.BlockSpec((5, 6), lambda i: (0, 0)),
        ],
        out_specs=[
            pl.BlockSpec((_BLK, 3), lambda i: (i, 0)),
            pl.BlockSpec((N_BATCH, 6), lambda i: (0, 0)),
            pl.BlockSpec((N_BATCH, 6), lambda i: (0, 0)),
            pl.BlockSpec((N_BATCH, 6), lambda i: (0, 0)),
        ],
        out_shape=[
            jax.ShapeDtypeStruct((N_ATOMS, 3), jnp.float32),
            jax.ShapeDtypeStruct((N_BATCH, 6), jnp.float32),
            jax.ShapeDtypeStruct((N_BATCH, 6), jnp.float32),
            jax.ShapeDtypeStruct((N_BATCH, 6), jnp.float32),
        ],
    )(fv, vv, batch3, iso, aniso, vol2, _A_T)


def kernel(edge_vec, edge_idx, pred_energy, atomic_stress_iso,
           atomic_stress_aniso, cell_volume, num_atoms, batch):
    src2 = edge_idx[0].reshape(N_GROUPS, G128)
    dst2 = edge_idx[1].reshape(N_GROUPS, G128)
    fv, vv = _sc_edge_scatter(edge_vec[:, 0], edge_vec[:, 1], edge_vec[:, 2],
                              src2, dst2)

    batch3 = batch.reshape(_NBLK, 1, _BLK)
    vol2 = cell_volume.reshape(N_BATCH, 1)
    force_drv, stress_drv, output_andev, output_dev = _tc_combine(
        fv, vv, batch3, atomic_stress_iso, atomic_stress_aniso, vol2)

    energy_out = jnp.squeeze(pred_energy, -1)
    return (energy_out, force_drv, stress_drv, output_andev, output_dev)


# speedup vs baseline: 32.8255x; 1.8744x over previous
import math

import jax
import jax.numpy as jnp
import numpy as np
from jax import lax
from jax.experimental import pallas as pl
from jax.experimental.pallas import tpu as pltpu
from jax.experimental.pallas import tpu_sc as plsc

N_ATOMS = 100000
N_EDGES = 3200000
N_BATCH = 16

NC = 2
NS = 16
NW = NC * NS

G128 = 128
CHUNK_G = 3
CHUNK = CHUNK_G * G128

N_CHUNKS_W = 261
N_EDGES_PAD = NW * N_CHUNKS_W * CHUNK
N_GROUPS_PAD = N_EDGES_PAD // G128

ROWS_A = 6248
ROWS_LAST = N_ATOMS - 15 * ROWS_A


def _sc_edge_body(evx, evy, evz, src2, dst2, z8, f_out, v_out,
                  f_acc, v_acc,
                  xb0, yb0, zb0, srcb0, dstb0, vec0, nvec0, vir0,
                  xb1, yb1, zb1, srcb1, dstb1, vec1, nvec1, vir1,
                  sem_i0, sem_i1, sem_s0, sem_s1, sem_x0, sem_x1):
    c = lax.axis_index("c")
    s = lax.axis_index("s")
    w = s * NC + c

    xb = (xb0, xb1)
    yb = (yb0, yb1)
    zb = (zb0, zb1)
    srcb = (srcb0, srcb1)
    dstb = (dstb0, dstb1)
    vecb = (vec0, vec1)
    nvecb = (nvec0, nvec1)
    virb = (vir0, vir1)
    sem_i = (sem_i0, sem_i1)
    sem_s = (sem_s0, sem_s1)
    sem_x = (sem_x0, sem_x1)

    r0 = s * ROWS_A
    pltpu.sync_copy(z8, f_acc.at[pl.ds(r0, ROWS_LAST), :])
    pltpu.sync_copy(z8, v_acc.at[pl.ds(r0, ROWS_LAST), :])
    for b in range(2):
        pltpu.sync_copy(z8.at[pl.ds(0, CHUNK), :], vecb[b])
        pltpu.sync_copy(z8.at[pl.ds(0, CHUNK), :], nvecb[b])
        pltpu.sync_copy(z8.at[pl.ds(0, CHUNK), :], virb[b])
    plsc.subcore_barrier()

    iota = lax.iota(jnp.int32, 16)
    cols = [jnp.full((16,), j, jnp.int32) for j in range(6)]
    neg1 = jnp.full((16,), -1.0, jnp.float32)
    ebase = w * N_CHUNKS_W * CHUNK

    def fire_in(b, k):
        e0 = ebase + k * CHUNK
        pltpu.async_copy(evx.at[pl.ds(e0, CHUNK)], xb[b], sem_i[b])
        pltpu.async_copy(evy.at[pl.ds(e0, CHUNK)], yb[b], sem_i[b])
        pltpu.async_copy(evz.at[pl.ds(e0, CHUNK)], zb[b], sem_i[b])

    def drain_in(b):
        pltpu.make_async_copy(evx.at[pl.ds(0, CHUNK)], xb[b], sem_i[b]).wait()
        pltpu.make_async_copy(evy.at[pl.ds(0, CHUNK)], yb[b], sem_i[b]).wait()
        pltpu.make_async_copy(evz.at[pl.ds(0, CHUNK)], zb[b], sem_i[b]).wait()

    def fire_idx(b, k):
        g0 = (ebase + k * CHUNK) // G128
        pltpu.async_copy(src2.at[pl.ds(g0, CHUNK_G), :], srcb[b], sem_x[b])
        pltpu.async_copy(dst2.at[pl.ds(g0, CHUNK_G), :], dstb[b], sem_x[b])

    def drain_idx(b):
        pltpu.make_async_copy(src2.at[pl.ds(0, CHUNK_G), :], srcb[b],
                              sem_x[b]).wait()
        pltpu.make_async_copy(dst2.at[pl.ds(0, CHUNK_G), :], dstb[b],
                              sem_x[b]).wait()

    def fire_s(b):
        for g in range(CHUNK_G):
            rows = pl.ds(g * G128, G128)
            pltpu.async_copy(vecb[b].at[rows, :], f_acc.at[srcb[b].at[g]],
                             sem_s[b], add=True)
            pltpu.async_copy(nvecb[b].at[rows, :], f_acc.at[dstb[b].at[g]],
                             sem_s[b], add=True)
            pltpu.async_copy(virb[b].at[rows, :], v_acc.at[dstb[b].at[g]],
                             sem_s[b], add=True)

    def drain_s(b):
        for _ in range(3 * CHUNK_G):
            pltpu.make_async_copy(z8.at[pl.ds(0, G128), :],
                                  vecb[b].at[pl.ds(0, G128), :],
                                  sem_s[b]).wait()

    def compute(b):
        def cbody(i, _):
            rows = i * 16 + iota
            x = plsc.load_gather(xb[b], [rows])
            y = plsc.load_gather(yb[b], [rows])
            z = plsc.load_gather(zb[b], [rows])
            plsc.store_scatter(vecb[b], [rows, cols[0]], x)
            plsc.store_scatter(vecb[b], [rows, cols[1]], y)
            plsc.store_scatter(vecb[b], [rows, cols[2]], z)
            plsc.store_scatter(nvecb[b], [rows, cols[0]], x * neg1)
            plsc.store_scatter(nvecb[b], [rows, cols[1]], y * neg1)
            plsc.store_scatter(nvecb[b], [rows, cols[2]], z * neg1)
            plsc.store_scatter(virb[b], [rows, cols[0]], x * x)
            plsc.store_scatter(virb[b], [rows, cols[1]], y * y)
            plsc.store_scatter(virb[b], [rows, cols[2]], z * z)
            plsc.store_scatter(virb[b], [rows, cols[3]], x * y)
            plsc.store_scatter(virb[b], [rows, cols[4]], y * z)
            plsc.store_scatter(virb[b], [rows, cols[5]], z * x)
            return _

        lax.fori_loop(0, CHUNK // 16, cbody, 0, unroll=4)

    fire_in(0, 0)
    fire_in(1, 1)

    def outer(m, carry):
        for b in range(2):
            k = 2 * m + b

            @pl.when(k < N_CHUNKS_W)
            def _():
                @pl.when(k >= 2)
                def _():
                    drain_s(b)

                fire_idx(b, k)
                drain_in(b)
                compute(b)
                drain_idx(b)
                fire_s(b)

                @pl.when(k + 2 < N_CHUNKS_W)
                def _():
                    fire_in(b, k + 2)

        return carry

    lax.fori_loop(0, (N_CHUNKS_W + 1) // 2, outer, 0)
    drain_s(0)
    drain_s(1)

    plsc.subcore_barrier()

    @pl.when(s < 15)
    def _():
        rows = pl.ds(s * ROWS_A, ROWS_A)
        pltpu.sync_copy(f_acc.at[rows, :], f_out.at[c, rows, :])
        pltpu.sync_copy(v_acc.at[rows, :], v_out.at[c, rows, :])

    @pl.when(s == 15)
    def _():
        rows = pl.ds(15 * ROWS_A, ROWS_LAST)
        pltpu.sync_copy(f_acc.at[rows, :], f_out.at[c, rows, :])
        pltpu.sync_copy(v_acc.at[rows, :], v_out.at[c, rows, :])


@jax.jit
def _sc_edge_scatter(evx, evy, evz, src2, dst2):
    z8 = jnp.zeros((ROWS_LAST, 8), jnp.float32)
    f = pl.kernel(
        _sc_edge_body,
        out_type=(
            jax.ShapeDtypeStruct((NC, N_ATOMS, 8), jnp.float32),
            jax.ShapeDtypeStruct((NC, N_ATOMS, 8), jnp.float32),
        ),
        mesh=plsc.VectorSubcoreMesh(
            core_axis_name="c", subcore_axis_name="s",
            num_cores=NC, num_subcores=NS),
        compiler_params=pltpu.CompilerParams(
            needs_layout_passes=False, use_tc_tiling_on_sc=False),
        scratch_types=[
            pltpu.VMEM_SHARED((N_ATOMS, 8), jnp.float32),
            pltpu.VMEM_SHARED((N_ATOMS, 8), jnp.float32),
        ] + 2 * [
            pltpu.VMEM((CHUNK,), jnp.float32),
            pltpu.VMEM((CHUNK,), jnp.float32),
            pltpu.VMEM((CHUNK,), jnp.float32),
            pltpu.VMEM((CHUNK_G, G128), jnp.int32),
            pltpu.VMEM((CHUNK_G, G128), jnp.int32),
            pltpu.VMEM((CHUNK, 8), jnp.float32),
            pltpu.VMEM((CHUNK, 8), jnp.float32),
            pltpu.VMEM((CHUNK, 8), jnp.float32),
        ] + 6 * [pltpu.SemaphoreType.DMA],
    )
    return f(evx, evy, evz, src2, dst2, z8)


_A_T = np.array([
    [0.0, 0.0, 0.0, 0.0, 0.0, 1 / math.sqrt(2)],
    [0.0, 0.0, 0.0, 1 / math.sqrt(2), 0.0, 0.0],
    [-1 / math.sqrt(6), math.sqrt(2) / math.sqrt(3), -1 / math.sqrt(6), 0.0, 0.0, 0.0],
    [0.0, 0.0, 0.0, 0.0, 1 / math.sqrt(2), 0.0],
    [-1 / math.sqrt(2), 0.0, 1 / math.sqrt(2), 0.0, 0.0, 0.0],
], dtype=np.float32)

_BLK = 1000
_NBLK = N_ATOMS // _BLK


def _tc_combine_body(fv, vv, batch3, iso, aniso, vol, a_t,
                     force_o, stress_o, andev_o, dev_o):
    i = pl.program_id(0)
    fvb = fv[...]
    vvb = vv[...]
    force_o[...] = fvb[0, :, 0:3] + fvb[1, :, 0:3]

    b = batch3[0]
    oh = (lax.broadcasted_iota(jnp.int32, (N_BATCH, _BLK), 0) == b
          ).astype(jnp.float32)
    sp = jnp.dot(oh, vvb[0] + vvb[1], preferred_element_type=jnp.float32)
    ap = jnp.dot(jnp.dot(oh, aniso[...], preferred_element_type=jnp.float32),
                 a_t[...], preferred_element_type=jnp.float32)
    ip = jnp.dot(oh, iso[...], preferred_element_type=jnp.float32)
    colmask = (lax.broadcasted_iota(jnp.int32, (N_BATCH, 6), 1) < 3
               ).astype(jnp.float32)

    @pl.when(i == 0)
    def _():
        stress_o[...] = jnp.zeros_like(stress_o)
        andev_o[...] = jnp.zeros_like(andev_o)
        dev_o[...] = jnp.zeros_like(dev_o)

    stress_o[...] += sp[:, 0:6]
    andev_o[...] += ap
    dev_o[...] += ip * colmask

    @pl.when(i == _NBLK - 1)
    def _():
        stress_o[...] = stress_o[...] * (-1.0 / vol[...])


@jax.jit
def _tc_combine(fv, vv, batch3, iso, aniso, vol2):
    return pl.pallas_call(
        _tc_combine_body,
        grid=(_NBLK,),
        in_specs=[
            pl.BlockSpec((NC, _BLK, 8), lambda i: (0, i, 0)),
            pl.BlockSpec((NC, _BLK, 8), lambda i: (0, i, 0)),
            pl.BlockSpec((1, 1, _BLK), lambda i: (i, 0, 0)),
            pl.BlockSpec((_BLK, 1), lambda i: (i, 0)),
            pl.BlockSpec((_BLK, 5), lambda i: (i, 0)),
            pl.BlockSpec((N_BATCH, 1), lambda i: (0, 0)),
            pl.BlockSpec((5, 6), lambda i: (0, 0)),
        ],
        out_specs=[
            pl.BlockSpec((_BLK, 3), lambda i: (i, 0)),
            pl.BlockSpec((N_BATCH, 6), lambda i: (0, 0)),
            pl.BlockSpec((N_BATCH, 6), lambda i: (0, 0)),
            pl.BlockSpec((N_BATCH, 6), lambda i: (0, 0)),
        ],
        out_shape=[
            jax.ShapeDtypeStruct((N_ATOMS, 3), jnp.float32),
            jax.ShapeDtypeStruct((N_BATCH, 6), jnp.float32),
            jax.ShapeDtypeStruct((N_BATCH, 6), jnp.float32),
            jax.ShapeDtypeStruct((N_BATCH, 6), jnp.float32),
        ],
    )(fv, vv, batch3, iso, aniso, vol2, _A_T)


def kernel(edge_vec, edge_idx, pred_energy, atomic_stress_iso,
           atomic_stress_aniso, cell_volume, num_atoms, batch):
    pad_e = N_EDGES_PAD - N_EDGES
    zpad = jnp.zeros((pad_e,), jnp.float32)
    ipad = jnp.zeros((pad_e,), edge_idx.dtype)
    src2 = jnp.concatenate([edge_idx[0], ipad]).reshape(N_GROUPS_PAD, G128)
    dst2 = jnp.concatenate([edge_idx[1], ipad]).reshape(N_GROUPS_PAD, G128)
    fv, vv = _sc_edge_scatter(
        jnp.concatenate([edge_vec[:, 0], zpad]),
        jnp.concatenate([edge_vec[:, 1], zpad]),
        jnp.concatenate([edge_vec[:, 2], zpad]),
        src2, dst2)

    batch3 = batch.reshape(_NBLK, 1, _BLK)
    vol2 = cell_volume.reshape(N_BATCH, 1)
    force_drv, stress_drv, output_andev, output_dev = _tc_combine(
        fv, vv, batch3, atomic_stress_iso, atomic_stress_aniso, vol2)

    energy_out = jnp.squeeze(pred_energy, -1)
    return (energy_out, force_drv, stress_drv, output_andev, output_dev)


# speedup vs baseline: 34.3154x; 1.0454x over previous
import math

import jax
import jax.numpy as jnp
import numpy as np
from jax import lax
from jax.experimental import pallas as pl
from jax.experimental.pallas import tpu as pltpu
from jax.experimental.pallas import tpu_sc as plsc

N_ATOMS = 100000
N_EDGES = 3200000
N_BATCH = 16

NC = 2
NS = 16
NW = NC * NS

G128 = 128
CHUNK_G = 3
CHUNK = CHUNK_G * G128

N_CHUNKS_W = 261
N_EDGES_PAD = NW * N_CHUNKS_W * CHUNK
N_GROUPS_PAD = N_EDGES_PAD // G128

ROWS_A = 6248
ROWS_LAST = N_ATOMS - 15 * ROWS_A


def _sc_edge_body(evx, evy, evz, src2, dst2, z8, f_out, v_out,
                  f_acc, v_acc,
                  xb0, yb0, zb0, srcb0, dstb0, vec0, nvec0, vir0,
                  xb1, yb1, zb1, srcb1, dstb1, vec1, nvec1, vir1,
                  sem_i0, sem_i1, sem_s0, sem_s1, sem_x0, sem_x1):
    c = lax.axis_index("c")
    s = lax.axis_index("s")
    w = s * NC + c

    xb = (xb0, xb1)
    yb = (yb0, yb1)
    zb = (zb0, zb1)
    srcb = (srcb0, srcb1)
    dstb = (dstb0, dstb1)
    vecb = (vec0, vec1)
    nvecb = (nvec0, nvec1)
    virb = (vir0, vir1)
    sem_i = (sem_i0, sem_i1)
    sem_s = (sem_s0, sem_s1)
    sem_x = (sem_x0, sem_x1)

    r0 = s * ROWS_A
    pltpu.sync_copy(z8, f_acc.at[pl.ds(r0, ROWS_LAST), :])
    pltpu.sync_copy(z8, v_acc.at[pl.ds(r0, ROWS_LAST), :])
    for b in range(2):
        pltpu.sync_copy(z8.at[pl.ds(0, CHUNK), :], vecb[b])
        pltpu.sync_copy(z8.at[pl.ds(0, CHUNK), :], nvecb[b])
        pltpu.sync_copy(z8.at[pl.ds(0, CHUNK), :], virb[b])
    plsc.subcore_barrier()

    iota = lax.iota(jnp.int32, 16)
    cols = [jnp.full((16,), j, jnp.int32) for j in range(6)]
    neg1 = jnp.full((16,), -1.0, jnp.float32)
    ebase = w * N_CHUNKS_W * CHUNK

    def fire_in(b, k):
        e0 = ebase + k * CHUNK
        pltpu.async_copy(evx.at[pl.ds(e0, CHUNK)], xb[b], sem_i[b])
        pltpu.async_copy(evy.at[pl.ds(e0, CHUNK)], yb[b], sem_i[b])
        pltpu.async_copy(evz.at[pl.ds(e0, CHUNK)], zb[b], sem_i[b])

    def drain_in(b):
        pltpu.make_async_copy(evx.at[pl.ds(0, CHUNK)], xb[b], sem_i[b]).wait()
        pltpu.make_async_copy(evy.at[pl.ds(0, CHUNK)], yb[b], sem_i[b]).wait()
        pltpu.make_async_copy(evz.at[pl.ds(0, CHUNK)], zb[b], sem_i[b]).wait()

    def fire_idx(b, k):
        g0 = (ebase + k * CHUNK) // G128
        pltpu.async_copy(src2.at[pl.ds(g0, CHUNK_G), :], srcb[b], sem_x[b])
        pltpu.async_copy(dst2.at[pl.ds(g0, CHUNK_G), :], dstb[b], sem_x[b])

    def drain_idx(b):
        pltpu.make_async_copy(src2.at[pl.ds(0, CHUNK_G), :], srcb[b],
                              sem_x[b]).wait()
        pltpu.make_async_copy(dst2.at[pl.ds(0, CHUNK_G), :], dstb[b],
                              sem_x[b]).wait()

    def fire_s(b):
        for g in range(CHUNK_G):
            rows = pl.ds(g * G128, G128)
            pltpu.async_copy(vecb[b].at[rows, :], f_acc.at[srcb[b].at[g]],
                             sem_s[b], add=True)
            pltpu.async_copy(nvecb[b].at[rows, :], f_acc.at[dstb[b].at[g]],
                             sem_s[b], add=True)
            pltpu.async_copy(virb[b].at[rows, :], v_acc.at[dstb[b].at[g]],
                             sem_s[b], add=True)

    def drain_s(b):
        for _ in range(3 * CHUNK_G):
            pltpu.make_async_copy(z8.at[pl.ds(0, G128), :],
                                  vecb[b].at[pl.ds(0, G128), :],
                                  sem_s[b]).wait()

    def compute(b):
        def cbody(i, _):
            rows = i * 16 + iota
            x = plsc.load_gather(xb[b], [rows])
            y = plsc.load_gather(yb[b], [rows])
            z = plsc.load_gather(zb[b], [rows])
            plsc.store_scatter(vecb[b], [rows, cols[0]], x)
            plsc.store_scatter(vecb[b], [rows, cols[1]], y)
            plsc.store_scatter(vecb[b], [rows, cols[2]], z)
            plsc.store_scatter(nvecb[b], [rows, cols[0]], x * neg1)
            plsc.store_scatter(nvecb[b], [rows, cols[1]], y * neg1)
            plsc.store_scatter(nvecb[b], [rows, cols[2]], z * neg1)
            plsc.store_scatter(virb[b], [rows, cols[0]], x * x)
            plsc.store_scatter(virb[b], [rows, cols[1]], y * y)
            plsc.store_scatter(virb[b], [rows, cols[2]], z * z)
            plsc.store_scatter(virb[b], [rows, cols[3]], x * y)
            plsc.store_scatter(virb[b], [rows, cols[4]], y * z)
            plsc.store_scatter(virb[b], [rows, cols[5]], z * x)
            return _

        lax.fori_loop(0, CHUNK // 16, cbody, 0, unroll=4)

    fire_in(0, 0)
    fire_in(1, 1)

    def outer(m, carry):
        for b in range(2):
            k = 2 * m + b

            @pl.when(k < N_CHUNKS_W)
            def _():
                @pl.when(k >= 2)
                def _():
                    drain_s(b)

                fire_idx(b, k)
                drain_in(b)
                compute(b)
                drain_idx(b)
                fire_s(b)

                @pl.when(k + 2 < N_CHUNKS_W)
                def _():
                    fire_in(b, k + 2)

        return carry

    lax.fori_loop(0, (N_CHUNKS_W + 1) // 2, outer, 0)
    drain_s(0)
    drain_s(1)

    plsc.subcore_barrier()

    @pl.when(s < 15)
    def _():
        rows = pl.ds(s * ROWS_A, ROWS_A)
        pltpu.sync_copy(f_acc.at[rows, :], f_out.at[c, rows, :])
        pltpu.sync_copy(v_acc.at[rows, :], v_out.at[c, rows, :])

    @pl.when(s == 15)
    def _():
        rows = pl.ds(15 * ROWS_A, ROWS_LAST)
        pltpu.sync_copy(f_acc.at[rows, :], f_out.at[c, rows, :])
        pltpu.sync_copy(v_acc.at[rows, :], v_out.at[c, rows, :])


@jax.jit
def _sc_edge_scatter(evx, evy, evz, src2, dst2):
    z8 = jnp.zeros((ROWS_LAST, 8), jnp.float32)
    f = pl.kernel(
        _sc_edge_body,
        out_type=(
            jax.ShapeDtypeStruct((NC, N_ATOMS, 8), jnp.float32),
            jax.ShapeDtypeStruct((NC, N_ATOMS, 8), jnp.float32),
        ),
        mesh=plsc.VectorSubcoreMesh(
            core_axis_name="c", subcore_axis_name="s",
            num_cores=NC, num_subcores=NS),
        compiler_params=pltpu.CompilerParams(
            needs_layout_passes=False, use_tc_tiling_on_sc=False),
        scratch_types=[
            pltpu.VMEM_SHARED((N_ATOMS, 8), jnp.float32),
            pltpu.VMEM_SHARED((N_ATOMS, 8), jnp.float32),
        ] + 2 * [
            pltpu.VMEM((CHUNK,), jnp.float32),
            pltpu.VMEM((CHUNK,), jnp.float32),
            pltpu.VMEM((CHUNK,), jnp.float32),
            pltpu.VMEM((CHUNK_G, G128), jnp.int32),
            pltpu.VMEM((CHUNK_G, G128), jnp.int32),
            pltpu.VMEM((CHUNK, 8), jnp.float32),
            pltpu.VMEM((CHUNK, 8), jnp.float32),
            pltpu.VMEM((CHUNK, 8), jnp.float32),
        ] + 6 * [pltpu.SemaphoreType.DMA],
    )
    return f(evx, evy, evz, src2, dst2, z8)


_A_T = np.array([
    [0.0, 0.0, 0.0, 0.0, 0.0, 1 / math.sqrt(2)],
    [0.0, 0.0, 0.0, 1 / math.sqrt(2), 0.0, 0.0],
    [-1 / math.sqrt(6), math.sqrt(2) / math.sqrt(3), -1 / math.sqrt(6), 0.0, 0.0, 0.0],
    [0.0, 0.0, 0.0, 0.0, 1 / math.sqrt(2), 0.0],
    [-1 / math.sqrt(2), 0.0, 1 / math.sqrt(2), 0.0, 0.0, 0.0],
], dtype=np.float32)

_BLK = 5000
_NBLK = N_ATOMS // _BLK


def _tc_combine_body(fv, vv, batch3, iso, aniso, vol, a_t,
                     force_o, stress_o, andev_o, dev_o):
    i = pl.program_id(0)
    fvb = fv[...]
    vvb = vv[...]
    force_o[...] = fvb[0, :, 0:3] + fvb[1, :, 0:3]

    b = batch3[0]
    oh = (lax.broadcasted_iota(jnp.int32, (N_BATCH, _BLK), 0) == b
          ).astype(jnp.float32)
    sp = jnp.dot(oh, vvb[0] + vvb[1], preferred_element_type=jnp.float32)
    ap = jnp.dot(jnp.dot(oh, aniso[...], preferred_element_type=jnp.float32),
                 a_t[...], preferred_element_type=jnp.float32)
    ip = jnp.dot(oh, iso[...], preferred_element_type=jnp.float32)
    colmask = (lax.broadcasted_iota(jnp.int32, (N_BATCH, 6), 1) < 3
               ).astype(jnp.float32)

    @pl.when(i == 0)
    def _():
        stress_o[...] = jnp.zeros_like(stress_o)
        andev_o[...] = jnp.zeros_like(andev_o)
        dev_o[...] = jnp.zeros_like(dev_o)

    stress_o[...] += sp[:, 0:6]
    andev_o[...] += ap
    dev_o[...] += ip * colmask

    @pl.when(i == _NBLK - 1)
    def _():
        stress_o[...] = stress_o[...] * (-1.0 / vol[...])


@jax.jit
def _tc_combine(fv, vv, batch3, iso, aniso, vol2):
    return pl.pallas_call(
        _tc_combine_body,
        grid=(_NBLK,),
        in_specs=[
            pl.BlockSpec((NC, _BLK, 8), lambda i: (0, i, 0)),
            pl.BlockSpec((NC, _BLK, 8), lambda i: (0, i, 0)),
            pl.BlockSpec((1, 1, _BLK), lambda i: (i, 0, 0)),
            pl.BlockSpec((_BLK, 1), lambda i: (i, 0)),
            pl.BlockSpec((_BLK, 5), lambda i: (i, 0)),
            pl.BlockSpec((N_BATCH, 1), lambda i: (0, 0)),
            pl.BlockSpec((5, 6), lambda i: (0, 0)),
        ],
        out_specs=[
            pl.BlockSpec((_BLK, 3), lambda i: (i, 0)),
            pl.BlockSpec((N_BATCH, 6), lambda i: (0, 0)),
            pl.BlockSpec((N_BATCH, 6), lambda i: (0, 0)),
            pl.BlockSpec((N_BATCH, 6), lambda i: (0, 0)),
        ],
        out_shape=[
            jax.ShapeDtypeStruct((N_ATOMS, 3), jnp.float32),
            jax.ShapeDtypeStruct((N_BATCH, 6), jnp.float32),
            jax.ShapeDtypeStruct((N_BATCH, 6), jnp.float32),
            jax.ShapeDtypeStruct((N_BATCH, 6), jnp.float32),
        ],
    )(fv, vv, batch3, iso, aniso, vol2, _A_T)


def kernel(edge_vec, edge_idx, pred_energy, atomic_stress_iso,
           atomic_stress_aniso, cell_volume, num_atoms, batch):
    pad_e = N_EDGES_PAD - N_EDGES
    zpad = jnp.zeros((pad_e,), jnp.float32)
    ipad = jnp.zeros((pad_e,), edge_idx.dtype)
    src2 = jnp.concatenate([edge_idx[0], ipad]).reshape(N_GROUPS_PAD, G128)
    dst2 = jnp.concatenate([edge_idx[1], ipad]).reshape(N_GROUPS_PAD, G128)
    fv, vv = _sc_edge_scatter(
        jnp.concatenate([edge_vec[:, 0], zpad]),
        jnp.concatenate([edge_vec[:, 1], zpad]),
        jnp.concatenate([edge_vec[:, 2], zpad]),
        src2, dst2)

    batch3 = batch.reshape(_NBLK, 1, _BLK)
    vol2 = cell_volume.reshape(N_BATCH, 1)
    force_drv, stress_drv, output_andev, output_dev = _tc_combine(
        fv, vv, batch3, atomic_stress_iso, atomic_stress_aniso, vol2)

    energy_out = jnp.squeeze(pred_energy, -1)
    return (energy_out, force_drv, stress_drv, output_andev, output_dev)


# speedup vs baseline: 35.1704x; 1.0249x over previous
import math

import jax
import jax.numpy as jnp
import numpy as np
from jax import lax
from jax.experimental import pallas as pl
from jax.experimental.pallas import tpu as pltpu
from jax.experimental.pallas import tpu_sc as plsc

N_ATOMS = 100000
N_EDGES = 3200000
N_BATCH = 16

NC = 2
NS = 16
NW = NC * NS

G128 = 128
CHUNK_G = 3
CHUNK = CHUNK_G * G128

N_CHUNKS_W = 261
N_EDGES_PAD = NW * N_CHUNKS_W * CHUNK
N_GROUPS_PAD = N_EDGES_PAD // G128

ROWS_A = 6248
ROWS_LAST = N_ATOMS - 15 * ROWS_A


def _sc_edge_body(evx, evy, evz, src2, dst2, z8, f_out, v_out,
                  f_acc, v_acc,
                  xb0, yb0, zb0, srcb0, dstb0, vec0, nvec0, vir0,
                  xb1, yb1, zb1, srcb1, dstb1, vec1, nvec1, vir1,
                  sem_i0, sem_i1, sem_s0, sem_s1, sem_x0, sem_x1):
    c = lax.axis_index("c")
    s = lax.axis_index("s")
    w = s * NC + c

    xb = (xb0, xb1)
    yb = (yb0, yb1)
    zb = (zb0, zb1)
    srcb = (srcb0, srcb1)
    dstb = (dstb0, dstb1)
    vecb = (vec0, vec1)
    nvecb = (nvec0, nvec1)
    virb = (vir0, vir1)
    sem_i = (sem_i0, sem_i1)
    sem_s = (sem_s0, sem_s1)
    sem_x = (sem_x0, sem_x1)

    r0 = s * ROWS_A
    pltpu.sync_copy(z8, f_acc.at[pl.ds(r0, ROWS_LAST), :])
    pltpu.sync_copy(z8, v_acc.at[pl.ds(r0, ROWS_LAST), :])
    for b in range(2):
        pltpu.sync_copy(z8.at[pl.ds(0, CHUNK), :], vecb[b])
        pltpu.sync_copy(z8.at[pl.ds(0, CHUNK), :], nvecb[b])
        pltpu.sync_copy(z8.at[pl.ds(0, CHUNK), :], virb[b])
    plsc.subcore_barrier()

    iota = lax.iota(jnp.int32, 16)
    cols = [jnp.full((16,), j, jnp.int32) for j in range(6)]
    neg1 = jnp.full((16,), -1.0, jnp.float32)
    ebase = w * N_CHUNKS_W * CHUNK

    def fire_in(b, k):
        e0 = ebase + k * CHUNK
        pltpu.async_copy(evx.at[pl.ds(e0, CHUNK)], xb[b], sem_i[b])
        pltpu.async_copy(evy.at[pl.ds(e0, CHUNK)], yb[b], sem_i[b])
        pltpu.async_copy(evz.at[pl.ds(e0, CHUNK)], zb[b], sem_i[b])

    def drain_in(b):
        pltpu.make_async_copy(evx.at[pl.ds(0, CHUNK)], xb[b], sem_i[b]).wait()
        pltpu.make_async_copy(evy.at[pl.ds(0, CHUNK)], yb[b], sem_i[b]).wait()
        pltpu.make_async_copy(evz.at[pl.ds(0, CHUNK)], zb[b], sem_i[b]).wait()

    def fire_idx(b, k):
        g0 = (ebase + k * CHUNK) // G128
        pltpu.async_copy(src2.at[pl.ds(g0, CHUNK_G), :], srcb[b], sem_x[b])
        pltpu.async_copy(dst2.at[pl.ds(g0, CHUNK_G), :], dstb[b], sem_x[b])

    def drain_idx(b):
        pltpu.make_async_copy(src2.at[pl.ds(0, CHUNK_G), :], srcb[b],
                              sem_x[b]).wait()
        pltpu.make_async_copy(dst2.at[pl.ds(0, CHUNK_G), :], dstb[b],
                              sem_x[b]).wait()

    def fire_s(b):
        for g in range(CHUNK_G):
            rows = pl.ds(g * G128, G128)
            pltpu.async_copy(vecb[b].at[rows, :], f_acc.at[srcb[b].at[g]],
                             sem_s[b], add=True)
            pltpu.async_copy(nvecb[b].at[rows, :], f_acc.at[dstb[b].at[g]],
                             sem_s[b], add=True)
            pltpu.async_copy(virb[b].at[rows, :], v_acc.at[dstb[b].at[g]],
                             sem_s[b], add=True)

    def drain_s(b):
        for _ in range(3 * CHUNK_G):
            pltpu.make_async_copy(z8.at[pl.ds(0, G128), :],
                                  vecb[b].at[pl.ds(0, G128), :],
                                  sem_s[b]).wait()

    def compute(b):
        def cbody(i, _):
            rows = i * 16 + iota
            x = plsc.load_gather(xb[b], [rows])
            y = plsc.load_gather(yb[b], [rows])
            z = plsc.load_gather(zb[b], [rows])
            plsc.store_scatter(vecb[b], [rows, cols[0]], x)
            plsc.store_scatter(vecb[b], [rows, cols[1]], y)
            plsc.store_scatter(vecb[b], [rows, cols[2]], z)
            plsc.store_scatter(nvecb[b], [rows, cols[0]], x * neg1)
            plsc.store_scatter(nvecb[b], [rows, cols[1]], y * neg1)
            plsc.store_scatter(nvecb[b], [rows, cols[2]], z * neg1)
            plsc.store_scatter(virb[b], [rows, cols[0]], x * x)
            plsc.store_scatter(virb[b], [rows, cols[1]], y * y)
            plsc.store_scatter(virb[b], [rows, cols[2]], z * z)
            plsc.store_scatter(virb[b], [rows, cols[3]], x * y)
            plsc.store_scatter(virb[b], [rows, cols[4]], y * z)
            plsc.store_scatter(virb[b], [rows, cols[5]], z * x)
            return _

        lax.fori_loop(0, CHUNK // 16, cbody, 0, unroll=4)

    fire_in(0, 0)
    fire_in(1, 1)

    def outer(m, carry):
        for b in range(2):
            k = 2 * m + b

            @pl.when(k < N_CHUNKS_W)
            def _():
                @pl.when(k >= 2)
                def _():
                    drain_s(b)

                fire_idx(b, k)
                drain_in(b)
                compute(b)
                drain_idx(b)
                fire_s(b)

                @pl.when(k + 2 < N_CHUNKS_W)
                def _():
                    fire_in(b, k + 2)

        return carry

    lax.fori_loop(0, (N_CHUNKS_W + 1) // 2, outer, 0)
    drain_s(0)
    drain_s(1)

    plsc.subcore_barrier()

    @pl.when(s < 15)
    def _():
        rows = pl.ds(s * ROWS_A, ROWS_A)
        pltpu.sync_copy(f_acc.at[rows, :], f_out.at[c, rows, :])
        pltpu.sync_copy(v_acc.at[rows, :], v_out.at[c, rows, :])

    @pl.when(s == 15)
    def _():
        rows = pl.ds(15 * ROWS_A, ROWS_LAST)
        pltpu.sync_copy(f_acc.at[rows, :], f_out.at[c, rows, :])
        pltpu.sync_copy(v_acc.at[rows, :], v_out.at[c, rows, :])


@jax.jit
def _sc_edge_scatter(evx, evy, evz, src2, dst2):
    z8 = jnp.zeros((ROWS_LAST, 8), jnp.float32)
    f = pl.kernel(
        _sc_edge_body,
        out_type=(
            jax.ShapeDtypeStruct((NC, N_ATOMS, 8), jnp.float32),
            jax.ShapeDtypeStruct((NC, N_ATOMS, 8), jnp.float32),
        ),
        mesh=plsc.VectorSubcoreMesh(
            core_axis_name="c", subcore_axis_name="s",
            num_cores=NC, num_subcores=NS),
        compiler_params=pltpu.CompilerParams(
            needs_layout_passes=False, use_tc_tiling_on_sc=False),
        scratch_types=[
            pltpu.VMEM_SHARED((N_ATOMS, 8), jnp.float32),
            pltpu.VMEM_SHARED((N_ATOMS, 8), jnp.float32),
        ] + 2 * [
            pltpu.VMEM((CHUNK,), jnp.float32),
            pltpu.VMEM((CHUNK,), jnp.float32),
            pltpu.VMEM((CHUNK,), jnp.float32),
            pltpu.VMEM((CHUNK_G, G128), jnp.int32),
            pltpu.VMEM((CHUNK_G, G128), jnp.int32),
            pltpu.VMEM((CHUNK, 8), jnp.float32),
            pltpu.VMEM((CHUNK, 8), jnp.float32),
            pltpu.VMEM((CHUNK, 8), jnp.float32),
        ] + 6 * [pltpu.SemaphoreType.DMA],
    )
    return f(evx, evy, evz, src2, dst2, z8)


_A_T = np.array([
    [0.0, 0.0, 0.0, 0.0, 0.0, 1 / math.sqrt(2)],
    [0.0, 0.0, 0.0, 1 / math.sqrt(2), 0.0, 0.0],
    [-1 / math.sqrt(6), math.sqrt(2) / math.sqrt(3), -1 / math.sqrt(6), 0.0, 0.0, 0.0],
    [0.0, 0.0, 0.0, 0.0, 1 / math.sqrt(2), 0.0],
    [-1 / math.sqrt(2), 0.0, 1 / math.sqrt(2), 0.0, 0.0, 0.0],
], dtype=np.float32)

_BLK = 5000
_NBLK = N_ATOMS // _BLK


def _tc_iso_body(batch3, iso, aniso, a_t, andev_o, dev_o):
    i = pl.program_id(0)
    b = batch3[0]
    oh = (lax.broadcasted_iota(jnp.int32, (N_BATCH, _BLK), 0) == b
          ).astype(jnp.float32)
    ap = jnp.dot(jnp.dot(oh, aniso[...], preferred_element_type=jnp.float32),
                 a_t[...], preferred_element_type=jnp.float32)
    ip = jnp.dot(oh, iso[...], preferred_element_type=jnp.float32)
    colmask = (lax.broadcasted_iota(jnp.int32, (N_BATCH, 6), 1) < 3
               ).astype(jnp.float32)

    @pl.when(i == 0)
    def _():
        andev_o[...] = jnp.zeros_like(andev_o)
        dev_o[...] = jnp.zeros_like(dev_o)

    andev_o[...] += ap
    dev_o[...] += ip * colmask


@jax.jit
def _tc_iso(batch3, iso, aniso):
    return pl.pallas_call(
        _tc_iso_body,
        grid=(_NBLK,),
        in_specs=[
            pl.BlockSpec((1, 1, _BLK), lambda i: (i, 0, 0)),
            pl.BlockSpec((_BLK, 1), lambda i: (i, 0)),
            pl.BlockSpec((_BLK, 5), lambda i: (i, 0)),
            pl.BlockSpec((5, 6), lambda i: (0, 0)),
        ],
        out_specs=[
            pl.BlockSpec((N_BATCH, 6), lambda i: (0, 0)),
            pl.BlockSpec((N_BATCH, 6), lambda i: (0, 0)),
        ],
        out_shape=[
            jax.ShapeDtypeStruct((N_BATCH, 6), jnp.float32),
            jax.ShapeDtypeStruct((N_BATCH, 6), jnp.float32),
        ],
    )(batch3, iso, aniso, _A_T)


def _tc_combine_body(fv, vv, batch3, vol, force_o, stress_o):
    i = pl.program_id(0)
    fvb = fv[...]
    vvb = vv[...]
    force_o[...] = fvb[0, :, 0:3] + fvb[1, :, 0:3]

    b = batch3[0]
    oh = (lax.broadcasted_iota(jnp.int32, (N_BATCH, _BLK), 0) == b
          ).astype(jnp.float32)
    sp = jnp.dot(oh, vvb[0] + vvb[1], preferred_element_type=jnp.float32)

    @pl.when(i == 0)
    def _():
        stress_o[...] = jnp.zeros_like(stress_o)

    stress_o[...] += sp[:, 0:6]

    @pl.when(i == _NBLK - 1)
    def _():
        stress_o[...] = stress_o[...] * (-1.0 / vol[...])


@jax.jit
def _tc_combine(fv, vv, batch3, vol2):
    return pl.pallas_call(
        _tc_combine_body,
        grid=(_NBLK,),
        in_specs=[
            pl.BlockSpec((NC, _BLK, 8), lambda i: (0, i, 0)),
            pl.BlockSpec((NC, _BLK, 8), lambda i: (0, i, 0)),
            pl.BlockSpec((1, 1, _BLK), lambda i: (i, 0, 0)),
            pl.BlockSpec((N_BATCH, 1), lambda i: (0, 0)),
        ],
        out_specs=[
            pl.BlockSpec((_BLK, 3), lambda i: (i, 0)),
            pl.BlockSpec((N_BATCH, 6), lambda i: (0, 0)),
        ],
        out_shape=[
            jax.ShapeDtypeStruct((N_ATOMS, 3), jnp.float32),
            jax.ShapeDtypeStruct((N_BATCH, 6), jnp.float32),
        ],
    )(fv, vv, batch3, vol2)


def kernel(edge_vec, edge_idx, pred_energy, atomic_stress_iso,
           atomic_stress_aniso, cell_volume, num_atoms, batch):
    pad_e = N_EDGES_PAD - N_EDGES
    zpad = jnp.zeros((pad_e,), jnp.float32)
    ipad = jnp.zeros((pad_e,), edge_idx.dtype)
    src2 = jnp.concatenate([edge_idx[0], ipad]).reshape(N_GROUPS_PAD, G128)
    dst2 = jnp.concatenate([edge_idx[1], ipad]).reshape(N_GROUPS_PAD, G128)
    fv, vv = _sc_edge_scatter(
        jnp.concatenate([edge_vec[:, 0], zpad]),
        jnp.concatenate([edge_vec[:, 1], zpad]),
        jnp.concatenate([edge_vec[:, 2], zpad]),
        src2, dst2)

    batch3 = batch.reshape(_NBLK, 1, _BLK)
    vol2 = cell_volume.reshape(N_BATCH, 1)
    output_andev, output_dev = _tc_iso(
        batch3, atomic_stress_iso, atomic_stress_aniso)
    force_drv, stress_drv = _tc_combine(fv, vv, batch3, vol2)

    energy_out = jnp.squeeze(pred_energy, -1)
    return (energy_out, force_drv, stress_drv, output_andev, output_dev)
